# baseline (speedup 1.0000x reference)
import jax
import jax.numpy as jnp
from jax.experimental import pallas as pl

_BB = 32


def _bcast_body(pe_ref, out_ref):
    out_ref[...] = jnp.broadcast_to(pe_ref[...][None, :, :], out_ref.shape)


def _bcast_part(pe_weight, nrows, name):
    max_len, d_model = pe_weight.shape
    f = pl.pallas_call(
        _bcast_body,
        grid=(nrows // _BB,),
        in_specs=[pl.BlockSpec((max_len, d_model), lambda i: (0, 0))],
        out_specs=pl.BlockSpec((_BB, max_len, d_model), lambda i: (i, 0, 0)),
        out_shape=jax.ShapeDtypeStruct((nrows, max_len, d_model), pe_weight.dtype),
        name=name,
    )
    return f(pe_weight)


def kernel(x, pe_weight):
    batch = x.shape[0]
    h = batch // 2
    a = _bcast_part(pe_weight, h, "bcast_a")
    b = _bcast_part(pe_weight, batch - h, "bcast_b")
    return jnp.concatenate([a, b], axis=0)


# speedup vs baseline: 3.0067x; 3.0067x over previous
import jax
import jax.numpy as jnp
from jax import lax
from jax.experimental import pallas as pl
from jax.experimental.pallas import tpu as pltpu

_SB = 8


def kernel(x, pe_weight):
    batch = x.shape[0]
    max_len, d_model = pe_weight.shape
    sb = _SB if batch % _SB == 0 else 1
    n_copies = batch // sb

    def body(pe_ref, out_ref, scratch_ref, sem):
        scratch_ref[...] = jnp.broadcast_to(
            pe_ref[...][None, :, :], scratch_ref.shape
        )

        def issue(i, carry):
            pltpu.make_async_copy(
                scratch_ref, out_ref.at[pl.ds(i * sb, sb)], sem
            ).start()
            return carry

        lax.fori_loop(0, n_copies, issue, 0, unroll=4)

        def drain(i, carry):
            pltpu.make_async_copy(
                scratch_ref, out_ref.at[pl.ds(i * sb, sb)], sem
            ).wait()
            return carry

        lax.fori_loop(0, n_copies, drain, 0, unroll=4)

    return pl.pallas_call(
        body,
        in_specs=[pl.BlockSpec(memory_space=pltpu.MemorySpace.VMEM)],
        out_specs=pl.BlockSpec(memory_space=pl.ANY),
        out_shape=jax.ShapeDtypeStruct((batch, max_len, d_model), pe_weight.dtype),
        scratch_shapes=[
            pltpu.VMEM((sb, max_len, d_model), pe_weight.dtype),
            pltpu.SemaphoreType.DMA,
        ],
    )(pe_weight)
